# Initial kernel scaffold; baseline (speedup 1.0000x reference)
import jax
import jax.numpy as jnp
from jax.experimental import pallas as pl


def kernel(hidden_states, weight, bias):
    raise NotImplementedError("write your pallas kernel here")


# speedup vs baseline: 1.1629x; 1.1629x over previous
import functools

import jax
import jax.numpy as jnp
from jax.experimental import pallas as pl
from jax.experimental.pallas import tpu as pltpu

E = 64
K = 8


def _router_block(x_ref, w_ref, b_ref, scores_ref, idx_ref):
    x = x_ref[...]
    w = w_ref[...]
    logits = jax.lax.dot_general(
        x, w, (((1,), (1,)), ((), ())),
        preferred_element_type=jnp.float32)
    logits = logits + b_ref[...][None, :]

    m = jnp.max(logits, axis=-1, keepdims=True)
    e = jnp.exp(logits - m)
    scores = e / jnp.sum(e, axis=-1, keepdims=True)
    scores_ref[...] = scores

    tb = scores.shape[0]
    iota = jax.lax.broadcasted_iota(jnp.int32, (tb, E), 1)
    s = scores
    neg = jnp.float32(-jnp.inf)
    for k in range(K):
        mk = jnp.max(s, axis=-1, keepdims=True)
        cand = jnp.where(s == mk, iota, E)
        amin = jnp.min(cand, axis=-1, keepdims=True)
        idx_ref[:, k] = amin[:, 0]
        s = jnp.where(iota == amin, neg, s)


@functools.partial(jax.jit, static_argnames=())
def kernel(hidden_states, weight, bias):
    Bn, Sn, Hn = hidden_states.shape
    T = Bn * Sn
    flat = hidden_states.reshape(T, Hn)
    TB = 512
    grid = (T // TB,)

    scores, idx = pl.pallas_call(
        _router_block,
        grid=grid,
        in_specs=[
            pl.BlockSpec((TB, Hn), lambda i: (i, 0)),
            pl.BlockSpec((E, Hn), lambda i: (0, 0)),
            pl.BlockSpec((E,), lambda i: (0,)),
        ],
        out_specs=[
            pl.BlockSpec((TB, E), lambda i: (i, 0)),
            pl.BlockSpec((TB, K), lambda i: (i, 0)),
        ],
        out_shape=[
            jax.ShapeDtypeStruct((T, E), jnp.float32),
            jax.ShapeDtypeStruct((T, K), jnp.int32),
        ],
        compiler_params=pltpu.CompilerParams(
            dimension_semantics=("arbitrary",),
        ),
    )(flat, weight, bias)
    return (scores, idx)


# speedup vs baseline: 1.3296x; 1.1433x over previous
import functools

import jax
import jax.numpy as jnp
from jax.experimental import pallas as pl
from jax.experimental.pallas import tpu as pltpu

E = 64
K = 8


def _router_block(x_ref, w_ref, b_ref, scores_ref, idx_ref):
    x = x_ref[...]
    w = w_ref[...]
    logits = jax.lax.dot_general(
        x, w, (((1,), (1,)), ((), ())),
        preferred_element_type=jnp.float32)
    logits = logits + b_ref[...][None, :]

    m = jnp.max(logits, axis=-1, keepdims=True)
    e = jnp.exp(logits - m)
    scores = e / jnp.sum(e, axis=-1, keepdims=True)
    scores_ref[...] = scores

    tb = scores.shape[0]
    iota = jax.lax.broadcasted_iota(jnp.int32, (tb, E), 1)
    s = scores
    neg = jnp.float32(-jnp.inf)
    for k in range(K):
        mk = jnp.max(s, axis=-1, keepdims=True)
        cand = jnp.where(s == mk, iota, E)
        amin = jnp.min(cand, axis=-1, keepdims=True)
        idx_ref[:, k] = amin[:, 0]
        s = jnp.where(iota == amin, neg, s)


@functools.partial(jax.jit, static_argnames=())
def kernel(hidden_states, weight, bias):
    Bn, Sn, Hn = hidden_states.shape
    T = Bn * Sn
    flat = hidden_states.reshape(T, Hn)
    TB = 1024
    grid = (T // TB,)

    scores, idx = pl.pallas_call(
        _router_block,
        grid=grid,
        in_specs=[
            pl.BlockSpec((TB, Hn), lambda i: (i, 0)),
            pl.BlockSpec((E, Hn), lambda i: (0, 0)),
            pl.BlockSpec((E,), lambda i: (0,)),
        ],
        out_specs=[
            pl.BlockSpec((TB, E), lambda i: (i, 0)),
            pl.BlockSpec((TB, K), lambda i: (i, 0)),
        ],
        out_shape=[
            jax.ShapeDtypeStruct((T, E), jnp.float32),
            jax.ShapeDtypeStruct((T, K), jnp.int32),
        ],
        compiler_params=pltpu.CompilerParams(
            dimension_semantics=("arbitrary",),
        ),
    )(flat, weight, bias)
    return (scores, idx)


# speedup vs baseline: 1.6803x; 1.2638x over previous
import jax
import jax.numpy as jnp
from jax.experimental import pallas as pl
from jax.experimental.pallas import tpu as pltpu

E = 64
K = 8


def _router_block(x_ref, w_ref, b_ref, scores_ref, idx_ref):
    x = x_ref[...]
    w = w_ref[...]
    lt = jax.lax.dot_general(
        w, x, (((1,), (1,)), ((), ())),
        preferred_element_type=jnp.float32)
    lt = lt + b_ref[...][:, None]

    m = jnp.max(lt, axis=0, keepdims=True)
    e = jnp.exp(lt - m)
    scores_t = e / jnp.sum(e, axis=0, keepdims=True)
    scores_ref[...] = scores_t.T

    tb = scores_t.shape[1]
    iota = jax.lax.broadcasted_iota(jnp.int32, (E, tb), 0)
    s = scores_t
    neg = jnp.float32(-jnp.inf)
    for k in range(K):
        mk = jnp.max(s, axis=0, keepdims=True)
        cand = jnp.where(s == mk, iota, E)
        amin = jnp.min(cand, axis=0, keepdims=True)
        idx_ref[k, :] = amin[0]
        s = jnp.where(iota == amin, neg, s)


def kernel(hidden_states, weight, bias):
    Bn, Sn, Hn = hidden_states.shape
    T = Bn * Sn
    flat = hidden_states.reshape(T, Hn)
    TB = 1024
    grid = (T // TB,)

    scores, idx_t = pl.pallas_call(
        _router_block,
        grid=grid,
        in_specs=[
            pl.BlockSpec((TB, Hn), lambda i: (i, 0)),
            pl.BlockSpec((E, Hn), lambda i: (0, 0)),
            pl.BlockSpec((E,), lambda i: (0,)),
        ],
        out_specs=[
            pl.BlockSpec((TB, E), lambda i: (i, 0)),
            pl.BlockSpec((K, TB), lambda i: (0, i)),
        ],
        out_shape=[
            jax.ShapeDtypeStruct((T, E), jnp.float32),
            jax.ShapeDtypeStruct((K, T), jnp.int32),
        ],
        compiler_params=pltpu.CompilerParams(
            dimension_semantics=("arbitrary",),
        ),
    )(flat, weight, bias)
    return (scores, idx_t.T)
